# baseline (speedup 1.0000x reference)
import functools

import jax
import jax.numpy as jnp
from jax import lax
from jax.experimental import pallas as pl
from jax.experimental.pallas import tpu as pltpu
from jax.experimental.pallas import tpu_sc as plsc

_F = 26
_V = 100000
_D = 32
_P = 128
_B = 16384
_R = _F * _D

_NC = 2
_NS = 16
_NW = _NC * _NS
_RPW = _R // _NW
_CHUNK = 8192
_NCHUNK = _B // _CHUNK
_H0 = 50048
_H1 = _V - _H0


def _make_gather():
    mesh = plsc.VectorSubcoreMesh(core_axis_name="c", subcore_axis_name="s")

    @functools.partial(
        pl.kernel,
        mesh=mesh,
        out_type=jax.ShapeDtypeStruct((_R, _B), jnp.float32),
        scratch_types=[
            pltpu.VMEM((_H0,), jnp.float32),
            pltpu.VMEM((_H1,), jnp.float32),
            pltpu.VMEM((_CHUNK,), jnp.int32),
            pltpu.VMEM((_B,), jnp.float32),
            pltpu.SemaphoreType.DMA,
            pltpu.SemaphoreType.DMA,
            pltpu.SemaphoreType.DMA,
            pltpu.SemaphoreType.DMA,
        ],
        compiler_params=pltpu.CompilerParams(
            use_tc_tiling_on_sc=True, needs_layout_passes=False),
    )
    def gather_k(idx_hbm, table_hbm, out_hbm, buf0_v, buf1_v, idx_v, out_v,
                 sem0, sem1, semo0, semo1):
        wid = lax.axis_index("s") * _NC + lax.axis_index("c")

        pltpu.sync_copy(table_hbm.at[wid, pl.ds(0, _H0)], buf0_v)

        def per_field(f, carry):
            r = f * _D + wid
            nxt = jnp.minimum(f + 1, _RPW - 1) * _D + wid
            cp1 = pltpu.async_copy(table_hbm.at[r, pl.ds(_H0, _H1)],
                                   buf1_v, sem1)

            def pass0_chunk(c, carry2):
                pltpu.sync_copy(idx_hbm.at[f, pl.ds(c * _CHUNK, _CHUNK)],
                                idx_v)

                def blk(j, carry3):
                    base = j * 128
                    for u in range(8):
                        o = base + u * 16
                        idx16 = idx_v[pl.ds(o, 16)]
                        ic = jnp.minimum(idx16, _H0 - 1)
                        out_v[pl.ds(c * _CHUNK + o, 16)] = (
                            plsc.load_gather(buf0_v, [ic]))
                    return carry3

                lax.fori_loop(0, _CHUNK // 128, blk, 0)
                return carry2

            lax.fori_loop(0, _NCHUNK, pass0_chunk, 0)
            cp1.wait()
            cp0 = pltpu.async_copy(table_hbm.at[nxt, pl.ds(0, _H0)],
                                   buf0_v, sem0)

            def pass1_chunk(c, carry2):
                pltpu.sync_copy(idx_hbm.at[f, pl.ds(c * _CHUNK, _CHUNK)],
                                idx_v)

                def blk(j, carry3):
                    base = j * 128
                    for u in range(8):
                        o = base + u * 16
                        idx16 = idx_v[pl.ds(o, 16)]
                        i1 = jnp.minimum(
                            jnp.maximum(idx16 - _H0, 0), _H1 - 1)
                        v1 = plsc.load_gather(buf1_v, [i1])
                        prev = out_v[pl.ds(c * _CHUNK + o, 16)]
                        out_v[pl.ds(c * _CHUNK + o, 16)] = jnp.where(
                            idx16 >= _H0, v1, prev)
                    return carry3

                lax.fori_loop(0, _CHUNK // 128, blk, 0)
                return carry2

            pass1_chunk(0, 0)
            cpo0 = pltpu.async_copy(out_v.at[pl.ds(0, _CHUNK)],
                                    out_hbm.at[r, pl.ds(0, _CHUNK)], semo0)
            pass1_chunk(1, 0)
            cpo1 = pltpu.async_copy(out_v.at[pl.ds(_CHUNK, _CHUNK)],
                                    out_hbm.at[r, pl.ds(_CHUNK, _CHUNK)],
                                    semo1)
            cp0.wait()
            cpo0.wait()
            cpo1.wait()
            return carry

        lax.fori_loop(0, _RPW, per_field, 0)

    return gather_k


_gather = _make_gather()

_BLK = 2048


def _gelu_exact(x):
    return 0.5 * x * (1.0 + lax.erf(x * 0.7071067811865476))


def _mlp_body(ht_ref, w1t_ref, b1_ref, w2t_ref, b2_ref, w3_ref, b3_ref, o_ref):
    ht = ht_ref[...]
    z = jnp.dot(w1t_ref[...], ht, preferred_element_type=jnp.float32)
    h1 = _gelu_exact(z + b1_ref[...])
    z = jnp.dot(w2t_ref[...], h1, preferred_element_type=jnp.float32)
    h2 = _gelu_exact(z + b2_ref[...])
    z = lax.dot_general(h2, w3_ref[...], (((0,), (0,)), ((), ())),
                        preferred_element_type=jnp.float32)
    o_ref[...] = z + b3_ref[...]


_mlp = pl.pallas_call(
    _mlp_body,
    grid=(_B // _BLK,),
    in_specs=[
        pl.BlockSpec((_R, _BLK), lambda i: (0, i)),
        pl.BlockSpec((_P, _R), lambda i: (0, 0)),
        pl.BlockSpec((_P, 1), lambda i: (0, 0)),
        pl.BlockSpec((_P, _P), lambda i: (0, 0)),
        pl.BlockSpec((_P, 1), lambda i: (0, 0)),
        pl.BlockSpec((_P, _P), lambda i: (0, 0)),
        pl.BlockSpec((1, _P), lambda i: (0, 0)),
    ],
    out_specs=pl.BlockSpec((_BLK, _P), lambda i: (i, 0)),
    out_shape=jax.ShapeDtypeStruct((_B, _P), jnp.float32),
)


def kernel(x_cat, tables, W1, b1, W2, b2, W3, b3):
    idx_fb = x_cat.T
    table_rows = tables.transpose(0, 2, 1).reshape(_R, _V)
    ht = _gather(idx_fb, table_rows)
    return _mlp(ht, W1.T, b1.reshape(_P, 1), W2.T, b2.reshape(_P, 1),
                W3, b3.reshape(1, _P))


# speedup vs baseline: 1.7050x; 1.7050x over previous
import functools

import jax
import jax.numpy as jnp
from jax import lax
from jax.experimental import pallas as pl
from jax.experimental.pallas import tpu as pltpu
from jax.experimental.pallas import tpu_sc as plsc

_F = 26
_V = 100000
_D = 32
_P = 128
_B = 16384
_R = _F * _D

_NC = 2
_NS = 16
_NW = _NC * _NS
_RPW = _R // _NW
_CHUNK = 8192
_NCHUNK = _B // _CHUNK


def _make_gather():
    mesh = plsc.VectorSubcoreMesh(core_axis_name="c", subcore_axis_name="s")

    @functools.partial(
        pl.kernel,
        mesh=mesh,
        out_type=jax.ShapeDtypeStruct((_R, _B), jnp.float32),
        scratch_types=[
            pltpu.VMEM((_V,), jnp.float32),
            pltpu.VMEM((_CHUNK,), jnp.int32),
            pltpu.VMEM((_CHUNK,), jnp.float32),
        ],
        compiler_params=pltpu.CompilerParams(
            use_tc_tiling_on_sc=True, needs_layout_passes=False),
    )
    def gather_k(idx_hbm, table_hbm, out_hbm, row_v, idx_v, out_v):
        wid = lax.axis_index("s") * _NC + lax.axis_index("c")

        def per_field(f, carry):
            r = f * _D + wid
            pltpu.sync_copy(table_hbm.at[r], row_v)

            def per_chunk(c, carry2):
                pltpu.sync_copy(idx_hbm.at[f, pl.ds(c * _CHUNK, _CHUNK)],
                                idx_v)

                @plsc.parallel_loop(0, _CHUNK // 16, unroll=8)
                def per_block(j):
                    o = j * 16
                    idx16 = idx_v[pl.ds(o, 16)]
                    out_v[pl.ds(o, 16)] = plsc.load_gather(row_v, [idx16])
                pltpu.sync_copy(out_v,
                                out_hbm.at[r, pl.ds(c * _CHUNK, _CHUNK)])
                return carry2

            lax.fori_loop(0, _NCHUNK, per_chunk, 0)
            return carry

        lax.fori_loop(0, _RPW, per_field, 0)

    return gather_k


_gather = _make_gather()

_BLK = 2048


def _gelu_exact(x):
    return 0.5 * x * (1.0 + lax.erf(x * 0.7071067811865476))


def _mlp_body(ht_ref, w1t_ref, b1_ref, w2t_ref, b2_ref, w3_ref, b3_ref, o_ref):
    ht = ht_ref[...]
    z = jnp.dot(w1t_ref[...], ht, preferred_element_type=jnp.float32)
    h1 = _gelu_exact(z + b1_ref[...])
    z = jnp.dot(w2t_ref[...], h1, preferred_element_type=jnp.float32)
    h2 = _gelu_exact(z + b2_ref[...])
    z = lax.dot_general(h2, w3_ref[...], (((0,), (0,)), ((), ())),
                        preferred_element_type=jnp.float32)
    o_ref[...] = z + b3_ref[...]


_mlp = pl.pallas_call(
    _mlp_body,
    grid=(_B // _BLK,),
    in_specs=[
        pl.BlockSpec((_R, _BLK), lambda i: (0, i)),
        pl.BlockSpec((_P, _R), lambda i: (0, 0)),
        pl.BlockSpec((_P, 1), lambda i: (0, 0)),
        pl.BlockSpec((_P, _P), lambda i: (0, 0)),
        pl.BlockSpec((_P, 1), lambda i: (0, 0)),
        pl.BlockSpec((_P, _P), lambda i: (0, 0)),
        pl.BlockSpec((1, _P), lambda i: (0, 0)),
    ],
    out_specs=pl.BlockSpec((_BLK, _P), lambda i: (i, 0)),
    out_shape=jax.ShapeDtypeStruct((_B, _P), jnp.float32),
)


def kernel(x_cat, tables, W1, b1, W2, b2, W3, b3):
    idx_fb = x_cat.T
    table_rows = tables.transpose(0, 2, 1).reshape(_R, _V)
    ht = _gather(idx_fb, table_rows)
    return _mlp(ht, W1.T, b1.reshape(_P, 1), W2.T, b2.reshape(_P, 1),
                W3, b3.reshape(1, _P))


# speedup vs baseline: 1.8006x; 1.0561x over previous
import functools

import jax
import jax.numpy as jnp
from jax import lax
from jax.experimental import pallas as pl
from jax.experimental.pallas import tpu as pltpu
from jax.experimental.pallas import tpu_sc as plsc

_F = 26
_V = 100000
_D = 32
_P = 128
_B = 16384
_R = _F * _D

_NC = 2
_NS = 16
_NW = _NC * _NS
_RPW = _R // _NW
_CHUNK = 8192
_NCHUNK = _B // _CHUNK


def _make_gather():
    mesh = plsc.VectorSubcoreMesh(core_axis_name="c", subcore_axis_name="s")

    @functools.partial(
        pl.kernel,
        mesh=mesh,
        out_type=jax.ShapeDtypeStruct((_R, _B), jnp.float32),
        scratch_types=[
            pltpu.VMEM((_V,), jnp.float32),
            pltpu.VMEM((_CHUNK,), jnp.int32),
            pltpu.VMEM((_CHUNK,), jnp.float32),
            pltpu.VMEM((_CHUNK,), jnp.float32),
            pltpu.SemaphoreType.DMA,
        ],
        compiler_params=pltpu.CompilerParams(
            use_tc_tiling_on_sc=True, needs_layout_passes=False),
    )
    def gather_k(idx_hbm, table_hbm, out_hbm, row_v, idx_v, out_v0, out_v1,
                 sem):
        wid = lax.axis_index("s") * _NC + lax.axis_index("c")
        c4 = _CHUNK * 4

        def do_field(f, needs_wait):
            r = f * _D + wid
            pltpu.sync_copy(table_hbm.at[r], row_v)

            for c, out_v in ((0, out_v0), (1, out_v1)):
                pltpu.sync_copy(idx_hbm.at[f, pl.ds(c * _CHUNK, _CHUNK)],
                                idx_v)
                if needs_wait:
                    pltpu.make_async_copy(
                        out_hbm.at[r, pl.ds(c * _CHUNK, _CHUNK)], out_v,
                        sem).wait()

                @plsc.parallel_loop(0, _CHUNK // 16, unroll=8)
                def per_block(j):
                    o = j * 16
                    idx16 = idx_v[pl.ds(o, 16)]
                    out_v[pl.ds(o, 16)] = plsc.load_gather(row_v, [idx16])
                pltpu.async_copy(out_v,
                                 out_hbm.at[r, pl.ds(c * _CHUNK, _CHUNK)],
                                 sem)

        do_field(0, False)
        lax.fori_loop(1, _RPW, lambda f, car: (do_field(f, True), car)[1], 0)
        pltpu.make_async_copy(out_hbm.at[wid, pl.ds(0, _CHUNK)], out_v0,
                              sem).wait()
        pltpu.make_async_copy(out_hbm.at[wid, pl.ds(0, _CHUNK)], out_v1,
                              sem).wait()

    return gather_k


_gather = _make_gather()

_BLK = 2048


def _gelu_exact(x):
    return 0.5 * x * (1.0 + lax.erf(x * 0.7071067811865476))


def _mlp_body(ht_ref, w1t_ref, b1_ref, w2t_ref, b2_ref, w3_ref, b3_ref, o_ref):
    ht = ht_ref[...]
    z = jnp.dot(w1t_ref[...], ht, preferred_element_type=jnp.float32)
    h1 = _gelu_exact(z + b1_ref[...])
    z = jnp.dot(w2t_ref[...], h1, preferred_element_type=jnp.float32)
    h2 = _gelu_exact(z + b2_ref[...])
    z = lax.dot_general(h2, w3_ref[...], (((0,), (0,)), ((), ())),
                        preferred_element_type=jnp.float32)
    o_ref[...] = z + b3_ref[...]


_mlp = pl.pallas_call(
    _mlp_body,
    grid=(_B // _BLK,),
    in_specs=[
        pl.BlockSpec((_R, _BLK), lambda i: (0, i)),
        pl.BlockSpec((_P, _R), lambda i: (0, 0)),
        pl.BlockSpec((_P, 1), lambda i: (0, 0)),
        pl.BlockSpec((_P, _P), lambda i: (0, 0)),
        pl.BlockSpec((_P, 1), lambda i: (0, 0)),
        pl.BlockSpec((_P, _P), lambda i: (0, 0)),
        pl.BlockSpec((1, _P), lambda i: (0, 0)),
    ],
    out_specs=pl.BlockSpec((_BLK, _P), lambda i: (i, 0)),
    out_shape=jax.ShapeDtypeStruct((_B, _P), jnp.float32),
)


def kernel(x_cat, tables, W1, b1, W2, b2, W3, b3):
    idx_fb = x_cat.T
    table_rows = tables.transpose(0, 2, 1).reshape(_R, _V)
    ht = _gather(idx_fb, table_rows)
    return _mlp(ht, W1.T, b1.reshape(_P, 1), W2.T, b2.reshape(_P, 1),
                W3, b3.reshape(1, _P))


# speedup vs baseline: 1.8629x; 1.0346x over previous
import functools

import jax
import jax.numpy as jnp
from jax import lax
from jax.experimental import pallas as pl
from jax.experimental.pallas import tpu as pltpu
from jax.experimental.pallas import tpu_sc as plsc

_F = 26
_V = 100000
_D = 32
_P = 128
_B = 16384
_R = _F * _D

_NC = 2
_NS = 16
_NW = _NC * _NS
_RPW = _R // _NW
_CHUNK = 4096
_NCHUNK = _B // _CHUNK


def _make_gather():
    mesh = plsc.VectorSubcoreMesh(core_axis_name="c", subcore_axis_name="s")

    @functools.partial(
        pl.kernel,
        mesh=mesh,
        out_type=jax.ShapeDtypeStruct((_R, _B), jnp.float32),
        scratch_types=[
            pltpu.VMEM((_V,), jnp.float32),
            pltpu.VMEM((_CHUNK,), jnp.int32),
            pltpu.VMEM((_CHUNK,), jnp.int32),
            pltpu.VMEM((_CHUNK,), jnp.float32),
            pltpu.VMEM((_CHUNK,), jnp.float32),
            pltpu.SemaphoreType.DMA,
            pltpu.SemaphoreType.DMA,
        ],
        compiler_params=pltpu.CompilerParams(
            use_tc_tiling_on_sc=True, needs_layout_passes=False),
    )
    def gather_k(idx_hbm, table_hbm, out_hbm, row_v, ib0, ib1, ob0, ob1,
                 sem_i, sem_o):
        wid = lax.axis_index("s") * _NC + lax.axis_index("c")
        ibufs = (ib0, ib1)
        obufs = (ob0, ob1)

        pltpu.async_copy(idx_hbm.at[0, pl.ds(0, _CHUNK)], ib0, sem_i)

        def do_field(f, out_waits):
            r = f * _D + wid
            pltpu.sync_copy(table_hbm.at[r], row_v)

            for c in range(_NCHUNK):
                ib = ibufs[c % 2]
                ob = obufs[c % 2]
                pltpu.make_async_copy(
                    idx_hbm.at[f, pl.ds(c * _CHUNK, _CHUNK)], ib,
                    sem_i).wait()
                if c + 1 < _NCHUNK:
                    nf, nc = f, c + 1
                else:
                    nf, nc = jnp.minimum(f + 1, _RPW - 1), 0
                pltpu.async_copy(
                    idx_hbm.at[nf, pl.ds(nc * _CHUNK, _CHUNK)],
                    ibufs[(c + 1) % 2], sem_i)
                if out_waits[c]:
                    pltpu.make_async_copy(
                        out_hbm.at[r, pl.ds(c * _CHUNK, _CHUNK)], ob,
                        sem_o).wait()

                @plsc.parallel_loop(0, _CHUNK // 16, unroll=8)
                def per_block(j):
                    o = j * 16
                    idx16 = ib[pl.ds(o, 16)]
                    ob[pl.ds(o, 16)] = plsc.load_gather(row_v, [idx16])
                pltpu.async_copy(ob,
                                 out_hbm.at[r, pl.ds(c * _CHUNK, _CHUNK)],
                                 sem_o)

        do_field(0, (False, False, True, True))
        lax.fori_loop(1, _RPW,
                      lambda f, car: (do_field(f, (True,) * _NCHUNK), car)[1],
                      0)
        pltpu.make_async_copy(out_hbm.at[wid, pl.ds(0, _CHUNK)], ob0,
                              sem_o).wait()
        pltpu.make_async_copy(out_hbm.at[wid, pl.ds(0, _CHUNK)], ob1,
                              sem_o).wait()
        pltpu.make_async_copy(idx_hbm.at[0, pl.ds(0, _CHUNK)], ib0,
                              sem_i).wait()

    return gather_k


_gather = _make_gather()

_BLK = 2048


def _gelu_exact(x):
    return 0.5 * x * (1.0 + lax.erf(x * 0.7071067811865476))


def _mlp_body(ht_ref, w1t_ref, b1_ref, w2t_ref, b2_ref, w3_ref, b3_ref, o_ref):
    ht = ht_ref[...]
    z = jnp.dot(w1t_ref[...], ht, preferred_element_type=jnp.float32)
    h1 = _gelu_exact(z + b1_ref[...])
    z = jnp.dot(w2t_ref[...], h1, preferred_element_type=jnp.float32)
    h2 = _gelu_exact(z + b2_ref[...])
    z = lax.dot_general(h2, w3_ref[...], (((0,), (0,)), ((), ())),
                        preferred_element_type=jnp.float32)
    o_ref[...] = z + b3_ref[...]


_mlp = pl.pallas_call(
    _mlp_body,
    grid=(_B // _BLK,),
    in_specs=[
        pl.BlockSpec((_R, _BLK), lambda i: (0, i)),
        pl.BlockSpec((_P, _R), lambda i: (0, 0)),
        pl.BlockSpec((_P, 1), lambda i: (0, 0)),
        pl.BlockSpec((_P, _P), lambda i: (0, 0)),
        pl.BlockSpec((_P, 1), lambda i: (0, 0)),
        pl.BlockSpec((_P, _P), lambda i: (0, 0)),
        pl.BlockSpec((1, _P), lambda i: (0, 0)),
    ],
    out_specs=pl.BlockSpec((_BLK, _P), lambda i: (i, 0)),
    out_shape=jax.ShapeDtypeStruct((_B, _P), jnp.float32),
)


def kernel(x_cat, tables, W1, b1, W2, b2, W3, b3):
    idx_fb = x_cat.T
    table_rows = tables.transpose(0, 2, 1).reshape(_R, _V)
    ht = _gather(idx_fb, table_rows)
    return _mlp(ht, W1.T, b1.reshape(_P, 1), W2.T, b2.reshape(_P, 1),
                W3, b3.reshape(1, _P))
